# Initial kernel scaffold; baseline (speedup 1.0000x reference)
import jax
import jax.numpy as jnp
from jax.experimental import pallas as pl


def kernel(feature):
    raise NotImplementedError("write your pallas kernel here")


# speedup vs baseline: 1.0380x; 1.0380x over previous
import jax
import jax.numpy as jnp
from jax.experimental import pallas as pl
from jax.experimental.pallas import tpu as pltpu

_N = 100000
_D = 64
_K = 10
_KP = 16
_R = 5000
_NB = _N // _R
_ITERS = 20
_RANDOM_STATE = 666


def _assign_body(x_ref, x2_ref, cT_ref, c2_ref, lab_ref, counts_ref):
    i = pl.program_id(0)
    kiota = jax.lax.broadcasted_iota(jnp.int32, (_R, _KP), 1)
    s = jax.lax.dot_general(x_ref[...], cT_ref[...], (((1,), (0,)), ((), ())),
                            preferred_element_type=jnp.float32)
    d = (x2_ref[...] + c2_ref[...]) - 2.0 * s
    d = jnp.where(kiota < _K, d, jnp.float32(1e30))
    vmin = jnp.min(d, axis=1, keepdims=True)
    lab = jnp.min(jnp.where(d == vmin, kiota, _KP), axis=1,
                  keepdims=True)
    lab_ref[...] = lab
    oh = (kiota == lab).astype(jnp.float32)
    cnt = jnp.sum(oh, axis=0, keepdims=True)

    @pl.when(i == 0)
    def _():
        counts_ref[...] = jnp.zeros_like(counts_ref)

    counts_ref[...] += cnt


def _assign(feature, x2, cT, c2row):
    return pl.pallas_call(
        _assign_body,
        grid=(_NB,),
        in_specs=[
            pl.BlockSpec((_R, _D), lambda i: (i, 0)),
            pl.BlockSpec((_R, 1), lambda i: (i, 0)),
            pl.BlockSpec((_D, _KP), lambda i: (0, 0)),
            pl.BlockSpec((1, _KP), lambda i: (0, 0)),
        ],
        out_specs=[
            pl.BlockSpec((_R, 1), lambda i: (i, 0)),
            pl.BlockSpec((1, _KP), lambda i: (0, 0)),
        ],
        out_shape=[
            jax.ShapeDtypeStruct((_N, 1), jnp.int32),
            jax.ShapeDtypeStruct((1, _KP), jnp.float32),
        ],
    )(feature, x2, cT, c2row)


def _select_min_body(lab_ref, dist_ref, val_ref, idx_ref):
    i = pl.program_id(0)
    kiota = jax.lax.broadcasted_iota(jnp.int32, (_R, _KP), 1)
    riota = jax.lax.broadcasted_iota(jnp.int32, (_R, _KP), 0)

    @pl.when(i == 0)
    def _():
        val_ref[...] = jnp.full_like(val_ref, jnp.inf)
        idx_ref[...] = jnp.zeros_like(idx_ref)

    masked = jnp.where(lab_ref[...] == kiota, dist_ref[...],
                       jnp.float32(jnp.inf))
    cmin = jnp.min(masked, axis=0, keepdims=True)
    cidx = jnp.min(jnp.where(masked == cmin, riota + i * _R, _N),
                   axis=0, keepdims=True)
    take = cmin < val_ref[...]
    idx_ref[...] = jnp.where(take, cidx, idx_ref[...])
    val_ref[...] = jnp.where(take, cmin, val_ref[...])


def _gather_body(x_ref, idx_ref, feat_ref):
    i = pl.program_id(0)
    riota = jax.lax.broadcasted_iota(jnp.int32, (_R, _KP), 0)

    @pl.when(i == 0)
    def _():
        feat_ref[...] = jnp.zeros_like(feat_ref)

    sel = (riota + i * _R == idx_ref[...]).astype(jnp.float32)
    feat_ref[...] += jax.lax.dot_general(
        sel, x_ref[...], (((0,), (0,)), ((), ())),
        preferred_element_type=jnp.float32)


def _select(feature, lab2d, dist):
    _, idx = pl.pallas_call(
        _select_min_body,
        grid=(_NB,),
        in_specs=[
            pl.BlockSpec((_R, 1), lambda i: (i, 0)),
            pl.BlockSpec((_R, 1), lambda i: (i, 0)),
        ],
        out_specs=[
            pl.BlockSpec((1, _KP), lambda i: (0, 0)),
            pl.BlockSpec((1, _KP), lambda i: (0, 0)),
        ],
        out_shape=[
            jax.ShapeDtypeStruct((1, _KP), jnp.float32),
            jax.ShapeDtypeStruct((1, _KP), jnp.int32),
        ],
    )(lab2d, dist)
    feat = pl.pallas_call(
        _gather_body,
        grid=(_NB,),
        in_specs=[
            pl.BlockSpec((_R, _D), lambda i: (i, 0)),
            pl.BlockSpec((1, _KP), lambda i: (0, 0)),
        ],
        out_specs=pl.BlockSpec((_KP, _D), lambda i: (0, 0)),
        out_shape=jax.ShapeDtypeStruct((_KP, _D), jnp.float32),
    )(feature, idx)
    return idx, feat


def _padded_centers(centers):
    cT = jnp.concatenate(
        [centers, jnp.zeros((_KP - _K, _D), jnp.float32)], axis=0).T
    c2 = jnp.sum(centers * centers, axis=1)
    c2row = jnp.concatenate([c2, jnp.zeros((_KP - _K,), jnp.float32)])[None, :]
    return cT, c2row


def kernel(feature):
    init_key = jax.random.key(_RANDOM_STATE)
    init_idx = jax.random.choice(init_key, _N, (_K,), replace=False)
    centers = feature[init_idx]
    x2 = jnp.sum(feature * feature, axis=1, keepdims=True)

    for _ in range(_ITERS):
        cT, c2row = _padded_centers(centers)
        lab2d, counts = _assign(feature, x2, cT, c2row)
        sums = jax.ops.segment_sum(feature, lab2d[:, 0], num_segments=_K)
        cnt = counts[0, :_K]
        centers = jnp.where(cnt[:, None] > 0,
                            sums / jnp.maximum(cnt, 1.0)[:, None], centers)

    cT, c2row = _padded_centers(centers)
    lab2d, _ = _assign(feature, x2, cT, c2row)
    diff = feature - centers[lab2d[:, 0]]
    dist = jnp.sqrt(jnp.sum(diff * diff, axis=1, keepdims=True))
    idx, feat = _select(feature, lab2d, dist)
    return centers, feat[:_K], idx[0, :_K]
